# baseline (speedup 1.0000x reference)
import functools

import jax
import jax.numpy as jnp
from jax import lax
from jax.experimental import pallas as pl
from jax.experimental.pallas import tpu as pltpu
from jax.experimental.pallas import tpu_sc as plsc

NUM_WORKERS = 32
R_ITEMS = 4
NBUF = 2
PPI = 100
PAD = 104


def _make_gather(batch, hist, d_model):
    per_w = batch // NUM_WORKERS
    n_it = per_w // (R_ITEMS * NBUF)
    assert per_w % (R_ITEMS * NBUF) == 0 and hist == 2 * PPI
    mesh = plsc.VectorSubcoreMesh(core_axis_name="c", subcore_axis_name="s")

    @functools.partial(
        pl.kernel,
        out_type=jax.ShapeDtypeStruct((batch, hist, d_model), jnp.float32),
        mesh=mesh,
        scratch_types=[
            pltpu.VMEM((NBUF, R_ITEMS, PAD), jnp.int32),
            pltpu.VMEM((NBUF, R_ITEMS * PAD, 2 * d_model), jnp.float32),
            pltpu.SemaphoreType.DMA,
            pltpu.SemaphoreType.DMA,
            pltpu.SemaphoreType.DMA,
        ],
        compiler_params=pltpu.CompilerParams(use_tc_tiling_on_sc=False),
    )
    def k(table_hbm, idx_hbm, out_hbm, idx_v, g_v, gsem, osem0, osem1):
        osems = (osem0, osem1)
        wid = lax.axis_index("s") * 2 + lax.axis_index("c")
        base = wid * per_w

        def drain_out(b, i0):
            for r in range(R_ITEMS):
                for half in range(2):
                    pltpu.make_async_copy(
                        g_v.at[b].at[pl.ds(r * PAD, PPI)].at[:, pl.ds(half * d_model, d_model)],
                        out_hbm.at[i0 + r].at[pl.ds(half * PPI, PPI)],
                        osems[b],
                    ).wait()

        def outer(t, carry):
            for b in range(NBUF):
                i0 = base + (t * NBUF + b) * R_ITEMS

                @pl.when(t > 0)
                def _wait_prev_scatter():
                    drain_out(b, base)

                pltpu.sync_copy(idx_hbm.at[pl.ds(i0, R_ITEMS)], idx_v.at[b])
                descs = [
                    pltpu.async_copy(
                        table_hbm.at[idx_v.at[b].at[r]],
                        g_v.at[b].at[pl.ds(r * PAD, PAD)],
                        gsem,
                    )
                    for r in range(R_ITEMS)
                ]
                for d in descs:
                    d.wait()
                for r in range(R_ITEMS):
                    for half in range(2):
                        pltpu.async_copy(
                            g_v.at[b].at[pl.ds(r * PAD, PPI)].at[:, pl.ds(half * d_model, d_model)],
                            out_hbm.at[i0 + r].at[pl.ds(half * PPI, PPI)],
                            osems[b],
                        )
            return carry

        lax.fori_loop(0, n_it, outer, 0)
        for b in range(NBUF):
            drain_out(b, base)

    return k


def kernel(cumulative_positions, position_embeddings):
    b, h = cumulative_positions.shape
    v = position_embeddings.shape[0]
    d = position_embeddings.shape[1]
    idx = cumulative_positions.astype(jnp.int32)
    pair_idx = idx[:, : h // 2] * v + idx[:, h // 2 :]
    pair_idx = jnp.pad(pair_idx, ((0, 0), (0, PAD - PPI)))
    left = jnp.broadcast_to(position_embeddings[:, None, :], (v, v, d))
    right = jnp.broadcast_to(position_embeddings[None, :, :], (v, v, d))
    t2 = jnp.concatenate([left, right], axis=-1).reshape(v * v, 2 * d)
    return _make_gather(b, h, d)(t2, pair_idx)
